# baseline (speedup 1.0000x reference)
import functools

import jax
import jax.numpy as jnp
from jax import lax
from jax.experimental import pallas as pl
from jax.experimental.pallas import tpu as pltpu
from jax.experimental.pallas import tpu_sc as plsc

NUM_SAMPLES_ = 32
NBUF = 10
CHUNK_H = 32


def _subsample_slices(c, t, h, w, ns):
    mesh = plsc.VectorSubcoreMesh(core_axis_name="c", subcore_axis_name="s")
    info = plsc.get_sparse_core_info()
    nc = info.num_cores
    nw = nc * info.num_subcores

    half = CHUNK_H
    per_slice = h // half
    slices_per_worker = (c * ns) // nw
    n_chunks = slices_per_worker * per_slice

    @functools.partial(
        pl.kernel,
        mesh=mesh,
        out_type=jax.ShapeDtypeStruct((c * ns, h, w), jnp.float32),
        scratch_types=[
            pltpu.VMEM((NBUF, half, w), jnp.float32),
            pltpu.SemaphoreType.DMA((NBUF,)),
            pltpu.SemaphoreType.DMA((NBUF,)),
        ],
    )
    def k(x_hbm, out_hbm, bufs, sem_g, sem_s):
        wid = lax.axis_index("s") * nc + lax.axis_index("c")
        r0 = wid * slices_per_worker

        def chunk(q):
            kk, hh = divmod(q, per_slice)
            r = r0 + kk
            cc = r // ns
            j = r % ns
            s = cc * t + (j * (t - 1)) // (ns - 1)
            return r, s, hh * half

        gathers = {}
        stores = {}

        def start_gather(q):
            i = q % NBUF
            _, s, h0 = chunk(q)
            gathers[q] = pltpu.async_copy(
                x_hbm.at[s, pl.ds(h0, half)], bufs.at[i], sem_g.at[i]
            )

        def start_store(q):
            i = q % NBUF
            r, _, h0 = chunk(q)
            stores[q] = pltpu.async_copy(
                bufs.at[i], out_hbm.at[r, pl.ds(h0, half)], sem_s.at[i]
            )

        for q in range(n_chunks + NBUF - 1):
            if q < n_chunks:
                if q >= NBUF:
                    stores.pop(q - NBUF).wait()
                start_gather(q)
            if q >= NBUF - 1:
                qq = q - (NBUF - 1)
                gathers.pop(qq).wait()
                start_store(qq)
        for qq in sorted(stores):
            stores.pop(qq).wait()

    return k


def kernel(x):
    c, t, h, w = x.shape
    ns = NUM_SAMPLES_
    x3 = x.reshape(c * t, h, w)
    out3 = _subsample_slices(c, t, h, w, ns)(x3)
    return out3.reshape(c, ns, h, w)


# speedup vs baseline: 1.0185x; 1.0185x over previous
import functools

import jax
import jax.numpy as jnp
from jax import lax
from jax.experimental import pallas as pl
from jax.experimental.pallas import tpu as pltpu
from jax.experimental.pallas import tpu_sc as plsc

NUM_SAMPLES_ = 32
NBUF = 6
CHUNK_H = 64


def _subsample_slices(c, t, h, w, ns):
    mesh = plsc.VectorSubcoreMesh(core_axis_name="c", subcore_axis_name="s")
    info = plsc.get_sparse_core_info()
    nc = info.num_cores
    nw = nc * info.num_subcores

    half = CHUNK_H
    per_slice = h // half
    slices_per_worker = (c * ns) // nw
    n_chunks = slices_per_worker * per_slice

    @functools.partial(
        pl.kernel,
        mesh=mesh,
        out_type=jax.ShapeDtypeStruct((c * ns, h, w), jnp.float32),
        scratch_types=[
            pltpu.VMEM((NBUF, half, w), jnp.float32),
            pltpu.SemaphoreType.DMA((NBUF,)),
            pltpu.SemaphoreType.DMA((NBUF,)),
        ],
    )
    def k(x_hbm, out_hbm, bufs, sem_g, sem_s):
        wid = lax.axis_index("s") * nc + lax.axis_index("c")
        r0 = wid * slices_per_worker

        def chunk(q):
            kk, hh = divmod(q, per_slice)
            r = r0 + kk
            cc = r // ns
            j = r % ns
            s = cc * t + (j * (t - 1)) // (ns - 1)
            return r, s, hh * half

        gathers = {}
        stores = {}

        def start_gather(q):
            i = q % NBUF
            _, s, h0 = chunk(q)
            gathers[q] = pltpu.async_copy(
                x_hbm.at[s, pl.ds(h0, half)], bufs.at[i], sem_g.at[i]
            )

        def start_store(q):
            i = q % NBUF
            r, _, h0 = chunk(q)
            stores[q] = pltpu.async_copy(
                bufs.at[i], out_hbm.at[r, pl.ds(h0, half)], sem_s.at[i]
            )

        for q in range(n_chunks + NBUF - 1):
            if q < n_chunks:
                if q >= NBUF:
                    stores.pop(q - NBUF).wait()
                start_gather(q)
            if q >= NBUF - 1:
                qq = q - (NBUF - 1)
                gathers.pop(qq).wait()
                start_store(qq)
        for qq in sorted(stores):
            stores.pop(qq).wait()

    return k


def kernel(x):
    c, t, h, w = x.shape
    ns = NUM_SAMPLES_
    x3 = x.reshape(c * t, h, w)
    out3 = _subsample_slices(c, t, h, w, ns)(x3)
    return out3.reshape(c, ns, h, w)


# speedup vs baseline: 1.0192x; 1.0007x over previous
import functools

import jax
import jax.numpy as jnp
from jax import lax
from jax.experimental import pallas as pl
from jax.experimental.pallas import tpu as pltpu
from jax.experimental.pallas import tpu_sc as plsc

NUM_SAMPLES_ = 32
NBUF = 7
CHUNK_H = 64


def _subsample_slices(c, t, h, w, ns):
    mesh = plsc.VectorSubcoreMesh(core_axis_name="c", subcore_axis_name="s")
    info = plsc.get_sparse_core_info()
    nc = info.num_cores
    nw = nc * info.num_subcores

    half = CHUNK_H
    per_slice = h // half
    slices_per_worker = (c * ns) // nw
    n_chunks = slices_per_worker * per_slice

    @functools.partial(
        pl.kernel,
        mesh=mesh,
        out_type=jax.ShapeDtypeStruct((c * ns, h, w), jnp.float32),
        scratch_types=[
            pltpu.VMEM((NBUF, half, w), jnp.float32),
            pltpu.SemaphoreType.DMA((NBUF,)),
            pltpu.SemaphoreType.DMA((NBUF,)),
        ],
    )
    def k(x_hbm, out_hbm, bufs, sem_g, sem_s):
        wid = lax.axis_index("s") * nc + lax.axis_index("c")
        r0 = wid * slices_per_worker

        def chunk(q):
            kk, hh = divmod(q, per_slice)
            r = r0 + kk
            cc = r // ns
            j = r % ns
            s = cc * t + (j * (t - 1)) // (ns - 1)
            return r, s, hh * half

        gathers = {}
        stores = {}

        def start_gather(q):
            i = q % NBUF
            _, s, h0 = chunk(q)
            gathers[q] = pltpu.async_copy(
                x_hbm.at[s, pl.ds(h0, half)], bufs.at[i], sem_g.at[i]
            )

        def start_store(q):
            i = q % NBUF
            r, _, h0 = chunk(q)
            stores[q] = pltpu.async_copy(
                bufs.at[i], out_hbm.at[r, pl.ds(h0, half)], sem_s.at[i]
            )

        for q in range(n_chunks + NBUF - 1):
            if q < n_chunks:
                if q >= NBUF:
                    stores.pop(q - NBUF).wait()
                start_gather(q)
            if q >= NBUF - 1:
                qq = q - (NBUF - 1)
                gathers.pop(qq).wait()
                start_store(qq)
        for qq in sorted(stores):
            stores.pop(qq).wait()

    return k


def kernel(x):
    c, t, h, w = x.shape
    ns = NUM_SAMPLES_
    x3 = x.reshape(c * t, h, w)
    out3 = _subsample_slices(c, t, h, w, ns)(x3)
    return out3.reshape(c, ns, h, w)
